# baseline (speedup 1.0000x reference)
import jax
import jax.numpy as jnp
from jax import lax
from jax.experimental import pallas as pl
from jax.experimental.pallas import tpu as pltpu
from jax.experimental.pallas import tpu_sc as plsc

NC = 2
NS = 16
NW = NC * NS
B = 16384
PER_W = B // NW
CHUNK = 128
NCHUNK = PER_W // CHUNK
L = 16


def _gather_body(table_hbm, idx_hbm, bias_hbm, out_hbm,
                 idx_v, rows_v, bias_v, sem, osem):
    cid = lax.axis_index("c")
    sid = lax.axis_index("s")
    wid = sid * NC + cid
    tab1d = table_hbm.at[0]
    idx_cp = pltpu.async_copy(idx_hbm.at[wid], idx_v, osem)
    bias_cp = pltpu.async_copy(bias_hbm, bias_v, osem)
    idx_cp.wait()
    copies = [
        pltpu.async_copy(tab1d.at[idx_v.at[j]], rows_v.at[j], sem)
        for j in range(NCHUNK)
    ]
    bias_cp.wait()
    bv = bias_v[...]
    outs = []
    for j in range(NCHUNK):
        copies[j].wait()
        for i in range(CHUNK // L):
            sl = pl.ds(i * L, L)
            rows_v[j, sl] = rows_v[j, sl] + bv
        outs.append(pltpu.async_copy(rows_v.at[j],
                                     out_hbm.at[wid * NCHUNK + j], osem))
    for o in outs:
        o.wait()


def kernel(inputs, categ_bias, output_layer_bias, moving_mean, moving_norm):
    idx = inputs[:, 0].astype(jnp.int32).reshape(NW, NCHUNK, CHUNK)
    table = jnp.swapaxes(categ_bias, 0, 1)
    bias16 = jnp.broadcast_to(output_layer_bias.reshape(1), (L,))
    run = pl.kernel(
        _gather_body,
        out_type=jax.ShapeDtypeStruct((NW * NCHUNK, CHUNK), jnp.float32),
        mesh=plsc.VectorSubcoreMesh(core_axis_name="c", subcore_axis_name="s"),
        scratch_types=[
            pltpu.VMEM((NCHUNK, CHUNK), jnp.int32),
            pltpu.VMEM((NCHUNK, CHUNK), jnp.float32),
            pltpu.VMEM((L,), jnp.float32),
            pltpu.SemaphoreType.DMA,
            pltpu.SemaphoreType.DMA,
        ],
    )
    out = run(table, idx, bias16)
    return out.reshape(B, 1)
